# baseline (speedup 1.0000x reference)
import functools

import jax
import jax.numpy as jnp
from jax import lax
from jax.experimental import pallas as pl
from jax.experimental.pallas import tpu as pltpu
from jax.experimental.pallas import tpu_sc as plsc

_B, _C, _H, _W = 64, 3, 224, 224
_BK = 16


@functools.partial(
    pl.kernel,
    mesh=plsc.VectorSubcoreMesh(core_axis_name="c", subcore_axis_name="s"),
    out_type=jax.ShapeDtypeStruct((_B,), jnp.float32),
    scratch_types=[
        pltpu.VMEM((_B,), jnp.int32),
        pltpu.VMEM((_B,), jnp.float32),
        pltpu.VMEM((_B,), jnp.float32),
    ],
)
def _sc_scale(p_hbm, m_hbm, out_hbm, pv, mv, sv):
    first = jnp.logical_and(lax.axis_index("c") == 0, lax.axis_index("s") == 0)

    @pl.when(first)
    def _():
        pltpu.sync_copy(p_hbm, pv)
        pltpu.sync_copy(m_hbm, mv)
        for i in range(_B // 16):
            sl = pl.ds(i * 16, 16)
            sv[sl] = jnp.where(pv[sl] != 0, mv[sl], jnp.float32(1.0))
        pltpu.sync_copy(sv, out_hbm)


def _mul_body(s_ref, x_ref, o_ref):
    i = pl.program_id(0)
    for j in range(_BK):
        o_ref[j] = x_ref[j] * s_ref[i * _BK + j]


def kernel(input, probs, magnitudes):
    scale = _sc_scale(probs.astype(jnp.int32), magnitudes)
    return pl.pallas_call(
        _mul_body,
        grid=(_B // _BK,),
        in_specs=[
            pl.BlockSpec(memory_space=pltpu.SMEM),
            pl.BlockSpec((_BK, _C, _H, _W), lambda i: (i, 0, 0, 0)),
        ],
        out_specs=pl.BlockSpec((_BK, _C, _H, _W), lambda i: (i, 0, 0, 0)),
        out_shape=jax.ShapeDtypeStruct((_B, _C, _H, _W), jnp.float32),
    )(scale, input)


# speedup vs baseline: 1.5824x; 1.5824x over previous
import jax
import jax.numpy as jnp
from jax.experimental import pallas as pl
from jax.experimental.pallas import tpu as pltpu

_B, _C, _H, _W = 64, 3, 224, 224
_ROWS = _C * _H * _W // 128


_BK = 16


def _scale_body(p_ref, m_ref, x_ref, o_ref):
    i = pl.program_id(0)
    for j in range(_BK):
        b = i * _BK + j
        scale = jnp.where(p_ref[b] != 0, m_ref[b], jnp.float32(1.0))
        o_ref[j] = x_ref[j] * scale


def _scale_body2(p_ref, m_ref, x_ref, o_ref):
    i = pl.program_id(0)
    for j in range(_BK):
        b = i * _BK + j
        scale = jnp.where(p_ref[b] != 0, m_ref[b], jnp.float32(1.0))
        o_ref[j] = x_ref[j] * scale


def kernel(input, probs, magnitudes):
    p = probs.astype(jnp.int32)
    return pl.pallas_call(
        _scale_body2,
        grid=(_B // _BK, 2),
        in_specs=[
            pl.BlockSpec(memory_space=pltpu.SMEM),
            pl.BlockSpec(memory_space=pltpu.SMEM),
            pl.BlockSpec((_BK, _C, _H // 2, _W), lambda i, k: (i, 0, k, 0)),
        ],
        out_specs=pl.BlockSpec((_BK, _C, _H // 2, _W), lambda i, k: (i, 0, k, 0)),
        out_shape=jax.ShapeDtypeStruct((_B, _C, _H, _W), jnp.float32),
    )(p, magnitudes, input)


# speedup vs baseline: 1.6044x; 1.0140x over previous
import jax
import jax.numpy as jnp
from jax.experimental import pallas as pl
from jax.experimental.pallas import tpu as pltpu

_B, _C, _H, _W = 64, 3, 224, 224
_CHUNK = 8
_RING = 4
_NSTEP = _B // _CHUNK


def _scale_body(p_ref, m_ref, x_hbm, o_hbm, xbuf, obuf, in_sem, out_sem):
    def rd(c):
        return pltpu.make_async_copy(
            x_hbm.at[pl.ds(c * _CHUNK, _CHUNK)],
            xbuf.at[c % _RING],
            in_sem.at[c % _RING],
        )

    def wr(c):
        return pltpu.make_async_copy(
            obuf.at[c % _RING],
            o_hbm.at[pl.ds(c * _CHUNK, _CHUNK)],
            out_sem.at[c % _RING],
        )

    for c in range(_RING - 1):
        rd(c).start()
    for c in range(_NSTEP):
        if c + _RING - 1 < _NSTEP:
            rd(c + _RING - 1).start()
        rd(c).wait()
        if c >= _RING:
            wr(c - _RING).wait()
        slot = c % _RING
        for j in range(_CHUNK):
            b = c * _CHUNK + j
            scale = jnp.where(p_ref[b] != 0, m_ref[b], jnp.float32(1.0))
            obuf[slot, j] = xbuf[slot, j] * scale
        wr(c).start()
    for c in range(_NSTEP - _RING, _NSTEP):
        wr(c).wait()


def kernel(input, probs, magnitudes):
    p = probs.astype(jnp.int32)
    return pl.pallas_call(
        _scale_body,
        in_specs=[
            pl.BlockSpec(memory_space=pltpu.SMEM),
            pl.BlockSpec(memory_space=pltpu.SMEM),
            pl.BlockSpec(memory_space=pl.ANY),
        ],
        out_specs=pl.BlockSpec(memory_space=pl.ANY),
        out_shape=jax.ShapeDtypeStruct((_B, _C, _H, _W), jnp.float32),
        scratch_shapes=[
            pltpu.VMEM((_RING, _CHUNK, _C, _H, _W), jnp.float32),
            pltpu.VMEM((_RING, _CHUNK, _C, _H, _W), jnp.float32),
            pltpu.SemaphoreType.DMA((_RING,)),
            pltpu.SemaphoreType.DMA((_RING,)),
        ],
    )(p, magnitudes, input)


# speedup vs baseline: 1.6248x; 1.0127x over previous
import jax
import jax.numpy as jnp
from jax.experimental import pallas as pl
from jax.experimental.pallas import tpu as pltpu

_B, _C, _H, _W = 64, 3, 224, 224
_CHUNK = 16
_RING = 3
_ORING = 2
_NSTEP = _B // _CHUNK


def _scale_body(p_ref, m_ref, x_hbm, o_hbm, xbuf, obuf, in_sem, out_sem):
    def rd(c):
        return pltpu.make_async_copy(
            x_hbm.at[pl.ds(c * _CHUNK, _CHUNK)],
            xbuf.at[c % _RING],
            in_sem.at[c % _RING],
        )

    def wr(c):
        return pltpu.make_async_copy(
            obuf.at[c % _ORING],
            o_hbm.at[pl.ds(c * _CHUNK, _CHUNK)],
            out_sem.at[c % _ORING],
        )

    for c in range(_RING - 1):
        rd(c).start()
    for c in range(_NSTEP):
        if c + _RING - 1 < _NSTEP:
            rd(c + _RING - 1).start()
        rd(c).wait()
        if c >= _ORING:
            wr(c - _ORING).wait()
        slot = c % _RING
        oslot = c % _ORING
        for j in range(_CHUNK):
            b = c * _CHUNK + j
            scale = jnp.where(p_ref[b] != 0, m_ref[b], jnp.float32(1.0))
            obuf[oslot, j] = xbuf[slot, j] * scale
        wr(c).start()
    for c in range(_NSTEP - _ORING, _NSTEP):
        wr(c).wait()


def kernel(input, probs, magnitudes):
    p = probs.astype(jnp.int32)
    return pl.pallas_call(
        _scale_body,
        in_specs=[
            pl.BlockSpec(memory_space=pltpu.SMEM),
            pl.BlockSpec(memory_space=pltpu.SMEM),
            pl.BlockSpec(memory_space=pl.ANY),
        ],
        out_specs=pl.BlockSpec(memory_space=pl.ANY),
        out_shape=jax.ShapeDtypeStruct((_B, _C, _H, _W), jnp.float32),
        scratch_shapes=[
            pltpu.VMEM((_RING, _CHUNK, _C, _H, _W), jnp.float32),
            pltpu.VMEM((_ORING, _CHUNK, _C, _H, _W), jnp.float32),
            pltpu.SemaphoreType.DMA((_RING,)),
            pltpu.SemaphoreType.DMA((_ORING,)),
        ],
    )(p, magnitudes, input)


# speedup vs baseline: 1.6829x; 1.0358x over previous
import jax
import jax.numpy as jnp
from jax.experimental import pallas as pl
from jax.experimental.pallas import tpu as pltpu

_B, _C, _H, _W = 64, 3, 224, 224
_ROWS = _C * _H * _W // 128


_BK = 16


def _scale_body(p_ref, m_ref, x_ref, o_ref):
    i = pl.program_id(0)
    for j in range(_BK):
        b = i * _BK + j
        scale = jnp.where(p_ref[b] != 0, m_ref[b], jnp.float32(1.0))
        o_ref[j] = x_ref[j] * scale


def kernel(input, probs, magnitudes):
    p = probs.astype(jnp.int32)
    return pl.pallas_call(
        _scale_body,
        grid=(_B // _BK,),
        in_specs=[
            pl.BlockSpec(memory_space=pltpu.SMEM),
            pl.BlockSpec(memory_space=pltpu.SMEM),
            pl.BlockSpec((_BK, _C, _H, _W), lambda i: (i, 0, 0, 0)),
        ],
        out_specs=pl.BlockSpec((_BK, _C, _H, _W), lambda i: (i, 0, 0, 0)),
        out_shape=jax.ShapeDtypeStruct((_B, _C, _H, _W), jnp.float32),
    )(p, magnitudes, input)
